# speedup vs baseline: 3.4728x; 3.4728x over previous
import functools
import math

import jax
import jax.numpy as jnp
import numpy as np
from jax import lax
from jax.experimental import pallas as pl
from jax.experimental.pallas import tpu as pltpu
from jax.experimental.pallas import tpu_sc as plsc

NC = 2
NS = 16
NW = NC * NS
L = 16

C = 16


@functools.lru_cache(maxsize=None)
def _pe_np(T: int, d_model: int):
    position = np.arange(T, dtype=np.float32)[:, None]
    div_term = np.exp(
        np.arange(0, d_model, 2, dtype=np.float32) * (-math.log(10000.0) / d_model)
    )
    pe = np.zeros((T, d_model), dtype=np.float32)
    pe[:, 0::2] = np.sin(position * div_term)
    if d_model % 2 == 1:
        pe[:, 1::2] = np.cos(position * div_term[:-1])
    else:
        pe[:, 1::2] = np.cos(position * div_term)
    return pe


@functools.lru_cache(maxsize=None)
def _build_sc_kernel(B: int, T: int, D: int, n_chunks: int):
    t_per_w = T // NW
    vregs_per_row = D // L

    mesh = plsc.VectorSubcoreMesh(core_axis_name="c", subcore_axis_name="s")

    @functools.partial(
        pl.kernel,
        out_type=jax.ShapeDtypeStruct((B, T, D), jnp.float32),
        mesh=mesh,
        scratch_types=[
            pltpu.VMEM((B * C,), jnp.int32),
            pltpu.VMEM((C, D), jnp.float32),
            pltpu.VMEM((B * C, D), jnp.float32),
            pltpu.SemaphoreType.DMA,
        ],
    )
    def k(ids_hbm, pe_hbm, table_hbm, out_hbm, idx_v, pe_v, rows_v, sem):
        w = lax.axis_index("s") * NC + lax.axis_index("c")
        t0 = w * t_per_w

        def chunk_body(c, _):
            tc0 = t0 + c * C
            pltpu.sync_copy(ids_hbm.at[w, c], idx_v)
            gather = pltpu.async_copy(table_hbm.at[idx_v], rows_v, sem)
            pltpu.sync_copy(pe_hbm.at[pl.ds(tc0, C)], pe_v)
            gather.wait()

            def add_body(v, _):
                for j in range(C):
                    pe_vec = pe_v[j, pl.ds(v * L, L)]
                    for b in range(B):
                        plsc.addupdate(
                            rows_v.at[b * C + j, pl.ds(v * L, L)], pe_vec
                        )
                return 0

            lax.fori_loop(0, vregs_per_row, add_body, 0, unroll=False)
            for b in range(B):
                pltpu.sync_copy(
                    rows_v.at[pl.ds(b * C, C)], out_hbm.at[b, pl.ds(tc0, C)]
                )
            return 0

        lax.fori_loop(0, n_chunks, chunk_body, 0, unroll=False)

    return k


def kernel(token_ids, token_emb_weight):
    B, T = token_ids.shape
    V, D = token_emb_weight.shape
    t_per_w = T // NW
    n_chunks = t_per_w // C

    ids = token_ids.astype(jnp.int32)
    ids_r = (
        ids.reshape(B, NW, n_chunks, C)
        .transpose(1, 2, 0, 3)
        .reshape(NW, n_chunks, B * C)
    )
    pe = jnp.asarray(_pe_np(T, D))
    k = _build_sc_kernel(B, T, D, n_chunks)
    return k(ids_r, pe, token_emb_weight)


# speedup vs baseline: 4.2802x; 1.2325x over previous
import functools
import math

import jax
import jax.numpy as jnp
import numpy as np
from jax import lax
from jax.experimental import pallas as pl
from jax.experimental.pallas import tpu as pltpu
from jax.experimental.pallas import tpu_sc as plsc

NC = 2
NS = 16
NW = NC * NS
L = 16

C = 8


@functools.lru_cache(maxsize=None)
def _pe_np(T: int, d_model: int):
    position = np.arange(T, dtype=np.float32)[:, None]
    div_term = np.exp(
        np.arange(0, d_model, 2, dtype=np.float32) * (-math.log(10000.0) / d_model)
    )
    pe = np.zeros((T, d_model), dtype=np.float32)
    pe[:, 0::2] = np.sin(position * div_term)
    if d_model % 2 == 1:
        pe[:, 1::2] = np.cos(position * div_term[:-1])
    else:
        pe[:, 1::2] = np.cos(position * div_term)
    return pe


@functools.lru_cache(maxsize=None)
def _build_sc_kernel(B: int, T: int, D: int, n_chunks: int):
    t_per_w = T // NW
    vregs_per_row = D // L

    mesh = plsc.VectorSubcoreMesh(core_axis_name="c", subcore_axis_name="s")

    @functools.partial(
        pl.kernel,
        out_type=jax.ShapeDtypeStruct((B, T, D), jnp.float32),
        mesh=mesh,
        scratch_types=[
            [pltpu.VMEM((B * C,), jnp.int32) for _ in range(2)],
            [pltpu.VMEM((C, D), jnp.float32) for _ in range(2)],
            [pltpu.VMEM((B * C, D), jnp.float32) for _ in range(2)],
            [pltpu.SemaphoreType.DMA for _ in range(2)],
            [pltpu.SemaphoreType.DMA for _ in range(2)],
            [pltpu.SemaphoreType.DMA for _ in range(2)],
        ],
    )
    def k(ids_hbm, pe_hbm, table_hbm, out_hbm, idx_v, pe_v, rows_v,
          gsem, psem, ssem):
        w = lax.axis_index("s") * NC + lax.axis_index("c")
        t0 = w * t_per_w

        gh = [None, None]
        ph = [None, None]
        sh = [None, None]

        def start(c):
            p = c % 2
            pltpu.sync_copy(ids_hbm.at[w, c], idx_v[p])
            gh[p] = pltpu.async_copy(table_hbm.at[idx_v[p]], rows_v[p], gsem[p])
            ph[p] = pltpu.async_copy(
                pe_hbm.at[pl.ds(t0 + c * C, C)], pe_v[p], psem[p]
            )

        start(0)
        for c in range(n_chunks):
            p = c % 2
            if c + 1 < n_chunks:
                if sh[1 - p] is not None:
                    for h in sh[1 - p]:
                        h.wait()
                    sh[1 - p] = None
                start(c + 1)
            gh[p].wait()
            ph[p].wait()

            def add_body(v, _, p=p):
                for j in range(C):
                    pe_vec = pe_v[p][j, pl.ds(v * L, L)]
                    for b in range(B):
                        plsc.addupdate(
                            rows_v[p].at[b * C + j, pl.ds(v * L, L)], pe_vec
                        )
                return 0

            lax.fori_loop(0, vregs_per_row, add_body, 0, unroll=False)
            tc0 = t0 + c * C
            sh[p] = [
                pltpu.async_copy(
                    rows_v[p].at[pl.ds(b * C, C)],
                    out_hbm.at[b, pl.ds(tc0, C)],
                    ssem[p],
                )
                for b in range(B)
            ]
        for p in range(2):
            if sh[p] is not None:
                for h in sh[p]:
                    h.wait()

    return k


def kernel(token_ids, token_emb_weight):
    B, T = token_ids.shape
    V, D = token_emb_weight.shape
    t_per_w = T // NW
    n_chunks = t_per_w // C

    ids = token_ids.astype(jnp.int32)
    ids_r = (
        ids.reshape(B, NW, n_chunks, C)
        .transpose(1, 2, 0, 3)
        .reshape(NW, n_chunks, B * C)
    )
    pe = jnp.asarray(_pe_np(T, D))
    k = _build_sc_kernel(B, T, D, n_chunks)
    return k(ids_r, pe, token_emb_weight)


# speedup vs baseline: 4.9543x; 1.1575x over previous
import functools
import math

import jax
import jax.numpy as jnp
import numpy as np
from jax import lax
from jax.experimental import pallas as pl
from jax.experimental.pallas import tpu as pltpu
from jax.experimental.pallas import tpu_sc as plsc

NC = 2
NS = 16
NW = NC * NS
L = 16

C = 8


@functools.lru_cache(maxsize=None)
def _pe_np(T: int, d_model: int):
    position = np.arange(T, dtype=np.float32)[:, None]
    div_term = np.exp(
        np.arange(0, d_model, 2, dtype=np.float32) * (-math.log(10000.0) / d_model)
    )
    pe = np.zeros((T, d_model), dtype=np.float32)
    pe[:, 0::2] = np.sin(position * div_term)
    if d_model % 2 == 1:
        pe[:, 1::2] = np.cos(position * div_term[:-1])
    else:
        pe[:, 1::2] = np.cos(position * div_term)
    return pe


@functools.lru_cache(maxsize=None)
def _build_sc_kernel(B: int, T: int, D: int, n_chunks: int):
    t_per_w = T // NW
    vregs_per_row = D // L

    mesh = plsc.VectorSubcoreMesh(core_axis_name="c", subcore_axis_name="s")

    @functools.partial(
        pl.kernel,
        out_type=jax.ShapeDtypeStruct((B, T, D), jnp.float32),
        mesh=mesh,
        scratch_types=[
            pltpu.VMEM((B, t_per_w), jnp.int32),
            [pltpu.VMEM((C, D), jnp.float32) for _ in range(2)],
            [pltpu.VMEM((B * C, D), jnp.float32) for _ in range(2)],
            [pltpu.SemaphoreType.DMA for _ in range(2)],
            [pltpu.SemaphoreType.DMA for _ in range(2)],
            [pltpu.SemaphoreType.DMA for _ in range(2)],
        ],
    )
    def k(ids_hbm, pe_hbm, table_hbm, out_hbm, ids_v, pe_v, rows_v,
          gsem, psem, ssem):
        w = lax.axis_index("s") * NC + lax.axis_index("c")
        t0 = w * t_per_w

        for b in range(B):
            pltpu.sync_copy(ids_hbm.at[b, pl.ds(t0, t_per_w)], ids_v.at[b])

        gh = [None, None]
        ph = [None, None]
        sh = [None, None]

        def start(c):
            p = c % 2
            gh[p] = [
                pltpu.async_copy(
                    table_hbm.at[ids_v.at[b, pl.ds(c * C, C)]],
                    rows_v[p].at[pl.ds(b * C, C)],
                    gsem[p],
                )
                for b in range(B)
            ]
            ph[p] = pltpu.async_copy(
                pe_hbm.at[pl.ds(t0 + c * C, C)], pe_v[p], psem[p]
            )

        start(0)
        for c in range(n_chunks):
            p = c % 2
            if c + 1 < n_chunks:
                if sh[1 - p] is not None:
                    for h in sh[1 - p]:
                        h.wait()
                    sh[1 - p] = None
                start(c + 1)
            for h in gh[p]:
                h.wait()
            ph[p].wait()

            def add_body(v, _, p=p):
                for j in range(C):
                    pe_vec = pe_v[p][j, pl.ds(v * L, L)]
                    for b in range(B):
                        plsc.addupdate(
                            rows_v[p].at[b * C + j, pl.ds(v * L, L)], pe_vec
                        )
                return 0

            lax.fori_loop(0, vregs_per_row, add_body, 0, unroll=False)
            tc0 = t0 + c * C
            sh[p] = [
                pltpu.async_copy(
                    rows_v[p].at[pl.ds(b * C, C)],
                    out_hbm.at[b, pl.ds(tc0, C)],
                    ssem[p],
                )
                for b in range(B)
            ]
        for p in range(2):
            if sh[p] is not None:
                for h in sh[p]:
                    h.wait()

    return k


def kernel(token_ids, token_emb_weight):
    B, T = token_ids.shape
    V, D = token_emb_weight.shape
    t_per_w = T // NW
    n_chunks = t_per_w // C

    ids = token_ids.astype(jnp.int32)
    pe = jnp.asarray(_pe_np(T, D))
    k = _build_sc_kernel(B, T, D, n_chunks)
    return k(ids, pe, token_emb_weight)
